# baseline (speedup 1.0000x reference)
import functools

import jax
import jax.numpy as jnp
from jax.experimental import pallas as pl
from jax.experimental.pallas import tpu as pltpu

_K = 16
_R2 = 16.0


def _pick_block(n, cap=512):
    best = 8
    for t in range(8, cap + 1, 8):
        if n % t == 0:
            best = t
    return best


def _conv_body(tgt_ref, posT_ref, srcM_ref, src_ref, W1f_ref, W1p_ref,
               b1_ref, W2_ref, b2_ref, W3_ref, b3_ref,
               pooled_ref, srccopy_ref, d2_ref, anyv_ref):
    tgt = tgt_ref[...]
    posT = posT_ref[...]
    T = tgt.shape[0]

    sn = jnp.sum(posT * posT, axis=0, keepdims=True)
    tn = jnp.sum(tgt * tgt, axis=1, keepdims=True)
    d2_ref[...] = sn - 2.0 * jax.lax.dot_general(
        tgt, posT, (((1,), (0,)), ((), ())),
        preferred_element_type=jnp.float32)

    def mm(a, b):
        return jax.lax.dot_general(a, b, (((1,), (0,)), ((), ())),
                                   preferred_element_type=jnp.float32)

    d_out = b3_ref.shape[-1]
    pooled_ref[...] = jnp.full((T, d_out), -jnp.inf, jnp.float32)
    anyv_ref[...] = jnp.zeros((T, 1), jnp.float32)

    def step(_, tok):
        d2p = d2_ref[...]
        m = jnp.min(d2p, axis=1, keepdims=True)
        oh = d2p == m
        d2_ref[...] = jnp.where(oh, jnp.inf, d2p)
        g = mm(oh.astype(jnp.float32), srcM_ref[...])
        feat = g[:, :128]
        rel = g[:, 128:131] - tgt
        h = jax.nn.relu(mm(feat, W1f_ref[...]) + mm(rel, W1p_ref[...])
                        + b1_ref[...])
        h = jax.nn.relu(mm(h, W2_ref[...]) + b2_ref[...])
        h = jax.nn.relu(mm(h, W3_ref[...]) + b3_ref[...])
        valid = (m + tn) <= _R2
        pooled_ref[...] = jnp.maximum(pooled_ref[...],
                                      jnp.where(valid, h, -jnp.inf))
        anyv_ref[...] = jnp.maximum(anyv_ref[...],
                                    valid.astype(jnp.float32))
        return tok

    jax.lax.fori_loop(0, _K, step, 0)
    pooled_ref[...] = jnp.where(anyv_ref[...] > 0.0, pooled_ref[...], 0.0)

    @pl.when(pl.program_id(0) == 0)
    def _():
        srccopy_ref[...] = src_ref[...]


def _pallas_forward(src, target, W1, b1, W2, b2, W3, b3, interpret=False):
    n_src, width = src.shape
    n_tgt = target.shape[0]
    T = _pick_block(n_tgt)
    grid = n_tgt // T

    posT = jnp.transpose(src[:, :3])
    srcM = jnp.concatenate([src[:, 3:], src[:, :3]], axis=1)
    W1f = W1[:128, :]
    W1p = W1[128:, :]
    b1r = b1.reshape(1, -1)
    b2r = b2.reshape(1, -1)
    b3r = b3.reshape(1, -1)
    d_out = W3.shape[1]

    fixed = lambda *shape: pl.BlockSpec(shape, lambda i: (0,) * len(shape))
    pooled, src_out = pl.pallas_call(
        _conv_body,
        grid=(grid,),
        in_specs=[
            pl.BlockSpec((T, 3), lambda i: (i, 0)),
            fixed(3, n_src),
            fixed(n_src, width),
            fixed(n_src, width),
            fixed(128, W1.shape[1]),
            fixed(width - 128, W1.shape[1]),
            fixed(1, b1.shape[0]),
            fixed(*W2.shape),
            fixed(1, b2.shape[0]),
            fixed(*W3.shape),
            fixed(1, b3.shape[0]),
        ],
        out_specs=[
            pl.BlockSpec((T, d_out), lambda i: (i, 0)),
            fixed(n_src, width),
        ],
        out_shape=[
            jax.ShapeDtypeStruct((n_tgt, d_out), jnp.float32),
            jax.ShapeDtypeStruct((n_src, width), jnp.float32),
        ],
        scratch_shapes=[pltpu.VMEM((T, n_src), jnp.float32),
                        pltpu.VMEM((T, 1), jnp.float32)],
        interpret=interpret,
    )(target, posT, srcM, src, W1f, W1p, b1r, W2, b2r, W3, b3r)
    return pooled, src_out


def kernel(src, target, W1, b1, W2, b2, W3, b3):
    _, src_out = _pallas_forward(src, target, W1, b1, W2, b2, W3, b3)
    return src_out
